# baseline (speedup 1.0000x reference)
import functools

import jax
import jax.numpy as jnp
from jax import lax
from jax.experimental import pallas as pl
from jax.experimental.pallas import tpu as pltpu
from jax.experimental.pallas import tpu_sc as plsc

B, L, V, S, D = 1024, 200, 100000, 2, 128

_info = plsc.get_sparse_core_info()
NC, NS, LN = _info.num_cores, _info.num_subcores, _info.num_lanes
NW = NC * NS
ROWS = B * L
RPW = ROWS // NW
G = 128
NG = RPW // G
KV = G // LN
L0S = tuple(sorted({(G * g) % L for g in range(NG)}))
LP = max((G * g) % L for g in range(NG)) + G

_mesh = plsc.VectorSubcoreMesh(core_axis_name="c", subcore_axis_name="s")

_DNUMS = lax.GatherDimensionNumbers(
    offset_dims=(), collapsed_slice_dims=(0,), start_index_map=(0,))


def _bcast_lane(vec, lane):
    idx = jnp.full((LN, 1), lane, jnp.int32)
    return lax.gather(vec, idx, dimension_numbers=_DNUMS, slice_sizes=(1,),
                      mode=lax.GatherScatterMode.PROMISE_IN_BOUNDS)


@functools.partial(
    pl.kernel,
    mesh=_mesh,
    out_type=jax.ShapeDtypeStruct((ROWS, D), jnp.float32),
    scratch_types=[
        pltpu.VMEM((NG, G), jnp.int32),
        pltpu.VMEM((NG, G), jnp.float32),
        pltpu.VMEM((NG, G), jnp.float32),
        pltpu.VMEM((LP, D), jnp.float32),
        pltpu.VMEM((1, D), jnp.float32),
        pltpu.VMEM((G, D), jnp.float32),
        pltpu.VMEM((G, D), jnp.float32),
        pltpu.VMEM((G, D), jnp.float32),
        pltpu.VMEM((G, D), jnp.float32),
        pltpu.SemaphoreType.DMA,
        pltpu.SemaphoreType.DMA,
        pltpu.SemaphoreType.DMA,
        pltpu.SemaphoreType.DMA,
    ],
)
def _emb_kernel(tok_hbm, pospad_hbm, seg1_hbm, tokens_hbm, segments_hbm,
                out_hbm, tokidx, maskf, segf, pospad, seg1, tb0, tb1,
                ob0, ob1, st0, st1, so0, so1):
    wid = lax.axis_index("s") * NC + lax.axis_index("c")
    base = wid * RPW
    bufs = ((tb0, ob0, st0, so0), (tb1, ob1, st1, so1))

    pltpu.sync_copy(tokens_hbm.at[wid], tokidx)
    pltpu.sync_copy(segments_hbm.at[wid], segf)
    pltpu.sync_copy(pospad_hbm, pospad)
    pltpu.sync_copy(seg1_hbm, seg1)

    def prep_body(gg, _):
        for kk in range(KV):
            off = kk * LN
            t16 = tokidx[gg, pl.ds(off, LN)]
            maskf[gg, pl.ds(off, LN)] = jnp.where(t16 == 0, 0.0, 1.0)
        return 0

    lax.fori_loop(0, NG, prep_body, 0)

    def issue_gather(g, b):
        tb = bufs[b][0]
        st = bufs[b][2]
        pltpu.async_copy(tok_hbm.at[tokidx.at[g]], tb, st)

    issue_gather(0, 0)
    issue_gather(1, 1)

    def pair_body(i, _):
        for b in range(2):
            g = i * 2 + b
            tb, ob, st, so = bufs[b]
            pltpu.make_async_copy(tok_hbm.at[tokidx.at[g]], tb, st).wait()

            @pl.when(g >= 2)
            def _drain():
                pltpu.make_async_copy(
                    ob, out_hbm.at[pl.ds(base + (g - 2) * G, G)], so).wait()

            l0 = lax.rem(g * G, L)
            sv = [seg1[0, pl.ds(c * LN, LN)] for c in range(D // LN)]
            pw = pospad.at[pl.ds(l0, G)]

            def add_block(jj, _):
                mt16 = maskf[g, pl.ds(jj * LN, LN)]
                ms16 = segf[g, pl.ds(jj * LN, LN)]
                for rr in range(LN):
                    mt = _bcast_lane(mt16, rr)
                    ms = _bcast_lane(ms16, rr)
                    for c in range(D // LN):
                        sl = pl.ds(c * LN, LN)
                        r = jj * LN + rr
                        ob[r, sl] = (tb[r, sl] * mt
                                     + (pw[r, sl] + sv[c] * ms))
                return 0

            lax.fori_loop(0, KV, add_block, 0)

            @pl.when(g + 2 < NG)
            def _next():
                issue_gather(g + 2, b)

            pltpu.async_copy(ob, out_hbm.at[pl.ds(base + g * G, G)], so)
        return 0

    lax.fori_loop(0, NG // 2, pair_body, 0)

    for b in range(2):
        g_last = NG - 2 + b
        ob, so = bufs[b][1], bufs[b][3]
        pltpu.make_async_copy(
            ob, out_hbm.at[pl.ds(base + g_last * G, G)], so).wait()


def kernel(tokens, segments, tok_table, pos_table, seg_table):
    pospad = jnp.concatenate([pos_table, pos_table[:LP - L]], axis=0)
    seg1 = seg_table[1][None, :]
    out = _emb_kernel(
        tok_table,
        pospad,
        seg1,
        tokens.reshape(NW, NG, G).astype(jnp.int32),
        segments.reshape(NW, NG, G).astype(jnp.float32),
    )
    return out.reshape(B, L, D)


# speedup vs baseline: 1.4863x; 1.4863x over previous
import functools

import jax
import jax.numpy as jnp
from jax import lax
from jax.experimental import pallas as pl
from jax.experimental.pallas import tpu as pltpu
from jax.experimental.pallas import tpu_sc as plsc

B, L, V, S, D = 1024, 200, 100000, 2, 128

_info = plsc.get_sparse_core_info()
NC, NS, LN = _info.num_cores, _info.num_subcores, _info.num_lanes
NW = NC * NS
ROWS = B * L
RPW = ROWS // NW
G = 80
NG = RPW // G
KV = G // LN
L0S = tuple(sorted({(G * g) % L for g in range(NG)}))
LP = max(L0S) + G

_mesh = plsc.VectorSubcoreMesh(core_axis_name="c", subcore_axis_name="s")

_DNUMS = lax.GatherDimensionNumbers(
    offset_dims=(), collapsed_slice_dims=(0,), start_index_map=(0,))


def _bcast_lane(vec, lane):
    idx = jnp.full((LN, 1), lane, jnp.int32)
    return lax.gather(vec, idx, dimension_numbers=_DNUMS, slice_sizes=(1,),
                      mode=lax.GatherScatterMode.PROMISE_IN_BOUNDS)


@functools.partial(
    pl.kernel,
    mesh=_mesh,
    out_type=jax.ShapeDtypeStruct((ROWS, D), jnp.float32),
    scratch_types=[
        pltpu.VMEM((NG, G), jnp.int32),
        pltpu.VMEM((NG, G), jnp.float32),
        pltpu.VMEM((NG, G), jnp.float32),
        pltpu.VMEM((LP, D), jnp.float32),
        pltpu.VMEM((1, D), jnp.float32),
        pltpu.VMEM((1, D), jnp.float32),
        pltpu.VMEM((G, D), jnp.float32),
        pltpu.VMEM((G, D), jnp.float32),
        pltpu.SemaphoreType.DMA,
        pltpu.SemaphoreType.DMA,
        pltpu.SemaphoreType.DMA,
        pltpu.SemaphoreType.DMA,
    ],
)
def _emb_kernel(tok_hbm, pospad_hbm, seg1_hbm, tok0_hbm, tokens_hbm,
                segments_hbm, out_hbm, tokidx, maskf, segf, pospad, seg1,
                tok0, tb0, tb1, st0, st1, so0, so1):
    wid = lax.axis_index("s") * NC + lax.axis_index("c")
    base = wid * RPW
    bufs = ((tb0, st0, so0), (tb1, st1, so1))

    pltpu.sync_copy(tokens_hbm.at[wid], tokidx)
    pltpu.sync_copy(segments_hbm.at[wid], segf)
    pltpu.sync_copy(pospad_hbm, pospad)
    pltpu.sync_copy(seg1_hbm, seg1)
    pltpu.sync_copy(tok0_hbm, tok0)

    def prep_body(gg, _):
        for kk in range(KV):
            off = kk * LN
            t16 = tokidx[gg, pl.ds(off, LN)]
            maskf[gg, pl.ds(off, LN)] = jnp.where(t16 == 0, -1.0, 0.0)
        return 0

    lax.fori_loop(0, NG, prep_body, 0)

    def issue_gather(g, b):
        tb, st, _ = bufs[b]
        pltpu.async_copy(tok_hbm.at[tokidx.at[g]], tb, st)

    issue_gather(0, 0)
    issue_gather(1, 1)

    def pair_body(i, _):
        for b in range(2):
            g = i * 2 + b
            tb, st, so = bufs[b]
            pltpu.make_async_copy(tok_hbm.at[tokidx.at[g]], tb, st).wait()

            l0 = lax.rem(g * G, L)
            sv = [seg1[0, pl.ds(c * LN, LN)] for c in range(D // LN)]
            tv = [tok0[0, pl.ds(c * LN, LN)] for c in range(D // LN)]

            for v in L0S:
                @pl.when(l0 == v)
                def _add_variant(v=v):
                    def add_block(jj, _):
                        mm16 = maskf[g, pl.ds(jj * LN, LN)]
                        ms16 = segf[g, pl.ds(jj * LN, LN)]
                        for rr in range(LN):
                            r = jj * LN + rr
                            mm = _bcast_lane(mm16, rr)
                            ms = _bcast_lane(ms16, rr)
                            for c in range(D // LN):
                                sl = pl.ds(c * LN, LN)
                                addend = (pospad[v + r, sl]
                                          + sv[c] * ms + tv[c] * mm)
                                plsc.addupdate(tb.at[r, sl], addend)
                        return 0

                    lax.fori_loop(0, KV, add_block, 0)

            pltpu.async_copy(tb, out_hbm.at[pl.ds(base + g * G, G)], so)
            pltpu.make_async_copy(
                tb, out_hbm.at[pl.ds(base + g * G, G)], so).wait()

            @pl.when(g + 2 < NG)
            def _next():
                issue_gather(g + 2, b)
        return 0

    lax.fori_loop(0, NG // 2, pair_body, 0)


def kernel(tokens, segments, tok_table, pos_table, seg_table):
    pospad = jnp.concatenate([pos_table, pos_table[:LP - L]], axis=0)
    seg1 = seg_table[1][None, :]
    tok0 = tok_table[0][None, :]
    out = _emb_kernel(
        tok_table,
        pospad,
        seg1,
        tok0,
        tokens.reshape(NW, NG, G).astype(jnp.int32),
        segments.reshape(NW, NG, G).astype(jnp.float32),
    )
    return out.reshape(B, L, D)


# speedup vs baseline: 1.8385x; 1.2370x over previous
import functools

import jax
import jax.numpy as jnp
from jax import lax
from jax.experimental import pallas as pl
from jax.experimental.pallas import tpu as pltpu
from jax.experimental.pallas import tpu_sc as plsc

B, L, V, S, D = 1024, 200, 100000, 2, 128

_info = plsc.get_sparse_core_info()
NC, NS, LN = _info.num_cores, _info.num_subcores, _info.num_lanes
NW = NC * NS
ROWS = B * L
RPW = ROWS // NW
G = L
NG = RPW // G
GA = 128
GB = G - GA
KF = G // LN
TAIL = G - KF * LN

_mesh = plsc.VectorSubcoreMesh(core_axis_name="c", subcore_axis_name="s")

_DNUMS = lax.GatherDimensionNumbers(
    offset_dims=(), collapsed_slice_dims=(0,), start_index_map=(0,))


def _bcast_lane(vec, lane):
    idx = jnp.full((LN, 1), lane, jnp.int32)
    return lax.gather(vec, idx, dimension_numbers=_DNUMS, slice_sizes=(1,),
                      mode=lax.GatherScatterMode.PROMISE_IN_BOUNDS)


@functools.partial(
    pl.kernel,
    mesh=_mesh,
    out_type=jax.ShapeDtypeStruct((ROWS, D), jnp.float32),
    scratch_types=[
        pltpu.VMEM((NG, G), jnp.int32),
        pltpu.VMEM((NG, LN), jnp.int32),
        pltpu.VMEM((NG, G), jnp.float32),
        pltpu.VMEM((L, D), jnp.float32),
        pltpu.VMEM((1, D), jnp.float32),
        pltpu.VMEM((1, D), jnp.float32),
        pltpu.VMEM((G, D), jnp.float32),
        pltpu.VMEM((G, D), jnp.float32),
        pltpu.SemaphoreType.DMA,
        pltpu.SemaphoreType.DMA,
        pltpu.SemaphoreType.DMA,
        pltpu.SemaphoreType.DMA,
    ],
)
def _emb_kernel(tok_hbm, pos_hbm, seg1_hbm, tok0_hbm, flags_hbm,
                tokens_hbm, segments_hbm, out_hbm, tokidx, flagsv, segf,
                posv, seg1, tok0, tb0, tb1, st0, st1, so0, so1):
    wid = lax.axis_index("s") * NC + lax.axis_index("c")
    base = wid * RPW
    bufs = ((tb0, st0, so0), (tb1, st1, so1))

    pltpu.sync_copy(tokens_hbm.at[wid], tokidx)
    pltpu.sync_copy(segments_hbm.at[wid], segf)
    pltpu.sync_copy(flags_hbm.at[wid], flagsv)
    pltpu.sync_copy(pos_hbm, posv)
    pltpu.sync_copy(seg1_hbm, seg1)
    pltpu.sync_copy(tok0_hbm, tok0)

    def issue_gathers(g, b):
        tb, st, _ = bufs[b]
        pltpu.async_copy(
            tok_hbm.at[tokidx.at[g, pl.ds(0, GA)]], tb.at[pl.ds(0, GA)], st)
        pltpu.async_copy(
            tok_hbm.at[tokidx.at[g, pl.ds(GA, GB)]], tb.at[pl.ds(GA, GB)], st)

    def wait_gathers(g, b):
        tb, st, _ = bufs[b]
        pltpu.make_async_copy(
            tok_hbm.at[tokidx.at[g, pl.ds(0, GA)]], tb.at[pl.ds(0, GA)],
            st).wait()
        pltpu.make_async_copy(
            tok_hbm.at[tokidx.at[g, pl.ds(GA, GB)]], tb.at[pl.ds(GA, GB)],
            st).wait()

    issue_gathers(0, 0)
    issue_gathers(1, 1)

    def pair_body(i, _):
        for b in range(2):
            g = i * 2 + b
            tb, st, so = bufs[b]
            wait_gathers(g, b)

            sv = [seg1[0, pl.ds(c * LN, LN)] for c in range(D // LN)]

            def add_block(jj, _):
                row0 = jj * LN
                ms16 = segf[g, pl.ds(row0, LN)]
                for rr in range(LN):
                    ms = _bcast_lane(ms16, rr)
                    r = row0 + rr
                    for c in range(D // LN):
                        sl = pl.ds(c * LN, LN)
                        tb[r, sl] = tb[r, sl] + (posv[r, sl] + sv[c] * ms)
                return 0

            lax.fori_loop(0, KF, add_block, 0)

            ms16t = segf[g, pl.ds(G - LN, LN)]
            for rr in range(TAIL):
                ms = _bcast_lane(ms16t, LN - TAIL + rr)
                r = KF * LN + rr
                for c in range(D // LN):
                    sl = pl.ds(c * LN, LN)
                    tb[r, sl] = tb[r, sl] + (posv[r, sl] + sv[c] * ms)

            has_pad = flagsv[g, pl.ds(0, LN)][0]

            @pl.when(has_pad != 0)
            def _pad_fix():
                tv = [tok0[0, pl.ds(c * LN, LN)] for c in range(D // LN)]

                def fix_block(jj, _):
                    row0 = jj * LN
                    t16 = tokidx[g, pl.ds(row0, LN)]
                    mm16 = jnp.where(t16 == 0, -1.0, 0.0)
                    for rr in range(LN):
                        mm = _bcast_lane(mm16, rr)
                        r = row0 + rr
                        for c in range(D // LN):
                            sl = pl.ds(c * LN, LN)
                            tb[r, sl] = tb[r, sl] + tv[c] * mm
                    return 0

                lax.fori_loop(0, KF, fix_block, 0)
                t16t = tokidx[g, pl.ds(G - LN, LN)]
                mm16t = jnp.where(t16t == 0, -1.0, 0.0)
                for rr in range(TAIL):
                    mm = _bcast_lane(mm16t, LN - TAIL + rr)
                    r = KF * LN + rr
                    for c in range(D // LN):
                        sl = pl.ds(c * LN, LN)
                        tb[r, sl] = tb[r, sl] + tv[c] * mm

            pltpu.async_copy(tb, out_hbm.at[pl.ds(base + g * G, G)], so)
            pltpu.make_async_copy(
                tb, out_hbm.at[pl.ds(base + g * G, G)], so).wait()

            @pl.when(g + 2 < NG)
            def _next():
                issue_gathers(g + 2, b)
        return 0

    lax.fori_loop(0, NG // 2, pair_body, 0)


def kernel(tokens, segments, tok_table, pos_table, seg_table):
    seg1 = seg_table[1][None, :]
    tok0 = tok_table[0][None, :]
    tok_r = tokens.reshape(NW, NG, G).astype(jnp.int32)
    flags = jnp.broadcast_to(
        jnp.any(tok_r == 0, axis=-1)[..., None], (NW, NG, LN)).astype(jnp.int32)
    out = _emb_kernel(
        tok_table,
        pos_table,
        seg1,
        tok0,
        flags,
        tok_r,
        segments.reshape(NW, NG, G).astype(jnp.float32),
    )
    return out.reshape(B, L, D)
